# Initial kernel scaffold; baseline (speedup 1.0000x reference)
import jax
import jax.numpy as jnp
from jax import lax
from jax.experimental import pallas as pl
from jax.experimental.pallas import tpu as pltpu
from jax.experimental.pallas import tpu_sc as plsc

VOCAB = 1000000
EMBED_DIM = 16
BATCH = 16384
NUM_FEATURES = 50

_INFO = plsc.get_sparse_core_info()
NC = _INFO.num_cores
NS = _INFO.num_subcores
NW = NC * NS
ROWS_PER_W = BATCH // NW
RB = 64
N_CHUNKS = ROWS_PER_W // RB
FLAT_PER_CHUNK = RB * NUM_FEATURES
K = 128
N_GATHERS = FLAT_PER_CHUNK // K


def _body(ids_hbm, rat_hbm, table_hbm, out_hbm, idx_v, rat_v, g_v, out_v, sem):
    wid = lax.axis_index("s") * NC + lax.axis_index("c")

    for c in range(N_CHUNKS):
        base_row = wid * ROWS_PER_W + c * RB
        idx_row0 = (wid * ROWS_PER_W // RB + c) * N_GATHERS

        pltpu.sync_copy(ids_hbm.at[pl.ds(idx_row0, N_GATHERS)], idx_v)
        pltpu.sync_copy(rat_hbm.at[pl.ds(base_row * NUM_FEATURES,
                                         FLAT_PER_CHUNK)], rat_v)

        copies = []
        for j in range(N_GATHERS):
            copies.append(
                pltpu.async_copy(table_hbm.at[idx_v.at[j]],
                                 g_v.at[pl.ds(j * K, K)], sem))
        for cp in copies:
            cp.wait()

        def row_body(i, _):
            base_f = i * NUM_FEATURES
            acc = jnp.zeros((EMBED_DIM,), jnp.float32)
            for n in range(NUM_FEATURES):
                r = rat_v[base_f + n]
                row = g_v[base_f + n, :]
                acc = acc + row * r
            out_v[i, :] = acc
            return 0

        lax.fori_loop(0, RB, row_body, 0)

        pltpu.sync_copy(out_v, out_hbm.at[pl.ds(base_row, RB)])


@jax.jit
def kernel(feature_ids, feature_ratings, table):
    ids2d = feature_ids.reshape(-1, K)
    rflat = feature_ratings.reshape(-1)

    mesh = plsc.VectorSubcoreMesh(core_axis_name="c", subcore_axis_name="s")
    f = pl.kernel(
        _body,
        out_type=jax.ShapeDtypeStruct((BATCH, EMBED_DIM), jnp.float32),
        mesh=mesh,
        scratch_types=[
            pltpu.VMEM((N_GATHERS, K), jnp.int32),
            pltpu.VMEM((FLAT_PER_CHUNK,), jnp.float32),
            pltpu.VMEM((FLAT_PER_CHUNK, EMBED_DIM), jnp.float32),
            pltpu.VMEM((RB, EMBED_DIM), jnp.float32),
            pltpu.SemaphoreType.DMA,
        ],
    )
    return f(ids2d, rflat, table)


# speedup vs baseline: 1.5817x; 1.5817x over previous
import jax
import jax.numpy as jnp
from jax import lax
from jax.experimental import pallas as pl
from jax.experimental.pallas import tpu as pltpu
from jax.experimental.pallas import tpu_sc as plsc

VOCAB = 1000000
EMBED_DIM = 16
BATCH = 16384
NUM_FEATURES = 50

_INFO = plsc.get_sparse_core_info()
NC = _INFO.num_cores
NS = _INFO.num_subcores
NW = NC * NS
ROWS_PER_W = BATCH // NW
RB = 64
N_CHUNKS = ROWS_PER_W // RB
FLAT_PER_CHUNK = RB * NUM_FEATURES
K = 128
N_GATHERS = FLAT_PER_CHUNK // K


def _body(ids_hbm, rat_hbm, table_hbm, out_hbm, idx_v, rat_v, g_v, out_v, sem):
    wid = lax.axis_index("s") * NC + lax.axis_index("c")

    for c in range(N_CHUNKS):
        base_row = wid * ROWS_PER_W + c * RB
        base_f = base_row * NUM_FEATURES

        pltpu.sync_copy(ids_hbm.at[pl.ds(base_f, FLAT_PER_CHUNK)], idx_v)
        pltpu.sync_copy(rat_hbm.at[pl.ds(base_f, FLAT_PER_CHUNK)], rat_v)

        copies = []
        for j in range(N_GATHERS):
            copies.append(
                pltpu.async_copy(table_hbm.at[idx_v.at[pl.ds(j * K, K)]],
                                 g_v.at[pl.ds(j * K, K)], sem))
        for cp in copies:
            cp.wait()

        def row_body(i, _):
            loc_f = i * NUM_FEATURES
            rv = [rat_v[pl.ds(loc_f, 16)],
                  rat_v[pl.ds(loc_f + 16, 16)],
                  rat_v[pl.ds(loc_f + 32, 16)],
                  rat_v[pl.ds(loc_f + 34, 16)]]
            acc = jnp.zeros((EMBED_DIM,), jnp.float32)
            for n in range(NUM_FEATURES):
                if n < 48:
                    r = rv[n // 16][n % 16]
                else:
                    r = rv[3][n - 34]
                row = g_v[loc_f + n, :]
                acc = acc + row * r
            out_v[i, :] = acc
            return 0

        lax.fori_loop(0, RB, row_body, 0)

        pltpu.sync_copy(out_v, out_hbm.at[pl.ds(base_row, RB)])


@jax.jit
def kernel(feature_ids, feature_ratings, table):
    iflat = feature_ids.reshape(-1)
    rflat = feature_ratings.reshape(-1)

    mesh = plsc.VectorSubcoreMesh(core_axis_name="c", subcore_axis_name="s")
    f = pl.kernel(
        _body,
        out_type=jax.ShapeDtypeStruct((BATCH, EMBED_DIM), jnp.float32),
        mesh=mesh,
        scratch_types=[
            pltpu.VMEM((FLAT_PER_CHUNK,), jnp.int32),
            pltpu.VMEM((FLAT_PER_CHUNK,), jnp.float32),
            pltpu.VMEM((FLAT_PER_CHUNK, EMBED_DIM), jnp.float32),
            pltpu.VMEM((RB, EMBED_DIM), jnp.float32),
            pltpu.SemaphoreType.DMA,
        ],
        compiler_params=pltpu.CompilerParams(use_tc_tiling_on_sc=False),
    )
    return f(iflat, rflat, table)


# speedup vs baseline: 1.6849x; 1.0652x over previous
import jax
import jax.numpy as jnp
from jax import lax
from jax.experimental import pallas as pl
from jax.experimental.pallas import tpu as pltpu
from jax.experimental.pallas import tpu_sc as plsc

VOCAB = 1000000
EMBED_DIM = 16
BATCH = 16384
NUM_FEATURES = 50

_INFO = plsc.get_sparse_core_info()
NC = _INFO.num_cores
NS = _INFO.num_subcores
NW = NC * NS
ROWS_PER_W = BATCH // NW
HB = 32
N_HALVES = ROWS_PER_W // HB
HFLAT = HB * NUM_FEATURES


def _body(ids_hbm, rat_hbm, table_hbm, out_hbm,
          ids_v, rat_v, g_v, out_v, sem0, sem1):
    wid = lax.axis_index("s") * NC + lax.axis_index("c")
    base_row = wid * ROWS_PER_W

    pltpu.sync_copy(ids_hbm.at[pl.ds(base_row, ROWS_PER_W)], ids_v)
    pltpu.sync_copy(rat_hbm.at[pl.ds(base_row, ROWS_PER_W)], rat_v)

    sems = (sem0, sem1)

    def fire_half(h, slot, sem):
        @pl.loop(0, HB)
        def _(loc):
            row = h * HB + loc
            pltpu.async_copy(
                table_hbm.at[ids_v.at[row]],
                g_v.at[pl.ds(slot * HFLAT + loc * NUM_FEATURES,
                             NUM_FEATURES)],
                sem)

    def drain_half(slot, sem):
        pltpu.make_async_copy(
            out_hbm.at[pl.ds(0, HFLAT)],
            g_v.at[pl.ds(slot * HFLAT, HFLAT)],
            sem).wait()

    def compute_half(h, slot):
        @pl.loop(0, HB)
        def _(loc):
            row = h * HB + loc
            loc_f = slot * HFLAT + loc * NUM_FEATURES
            rv = [rat_v[row, pl.ds(0, 16)],
                  rat_v[row, pl.ds(16, 16)],
                  rat_v[row, pl.ds(32, 16)],
                  rat_v[row, pl.ds(34, 16)]]
            acc = jnp.zeros((EMBED_DIM,), jnp.float32)
            for n in range(NUM_FEATURES):
                if n < 48:
                    r = rv[n // 16][n % 16]
                else:
                    r = rv[3][n - 34]
                acc = acc + g_v[loc_f + n, :] * r
            out_v[row, :] = acc

    fire_half(0, 0, sems[0])

    @pl.loop(0, N_HALVES, step=2)
    def _(h):
        fire_half(h + 1, 1, sems[1])
        drain_half(0, sems[0])
        compute_half(h, 0)

        @pl.when(h + 2 < N_HALVES)
        def _():
            fire_half(h + 2, 0, sems[0])
        drain_half(1, sems[1])
        compute_half(h + 1, 1)

    pltpu.sync_copy(out_v, out_hbm.at[pl.ds(base_row, ROWS_PER_W)])


@jax.jit
def kernel(feature_ids, feature_ratings, table):
    mesh = plsc.VectorSubcoreMesh(core_axis_name="c", subcore_axis_name="s")
    f = pl.kernel(
        _body,
        out_type=jax.ShapeDtypeStruct((BATCH, EMBED_DIM), jnp.float32),
        mesh=mesh,
        scratch_types=[
            pltpu.VMEM((ROWS_PER_W, NUM_FEATURES), jnp.int32),
            pltpu.VMEM((ROWS_PER_W, NUM_FEATURES), jnp.float32),
            pltpu.VMEM((2 * HFLAT, EMBED_DIM), jnp.float32),
            pltpu.VMEM((ROWS_PER_W, EMBED_DIM), jnp.float32),
            pltpu.SemaphoreType.DMA,
            pltpu.SemaphoreType.DMA,
        ],
        compiler_params=pltpu.CompilerParams(use_tc_tiling_on_sc=False),
    )
    return f(feature_ids, feature_ratings, table)
